# baseline (speedup 1.0000x reference)
import jax
import jax.numpy as jnp
from jax import lax
from jax.experimental import pallas as pl
from jax.experimental.pallas import tpu as pltpu

_B = 32
_STATE = 256
_HID = 1024
_F = 1000
_V = 64
_RTOT = _V * _F
_BLK = 1280
_GRID = _RTOT // _BLK

_CONTRACT_LAST = (((1,), (1,)), ((), ()))


def _body(s_ref, w1_ref, b1_ref, w2_ref, b2_ref, wc_ref, bc_ref,
          wr_ref, br_ref, cache_ref, rec_ref, h_ref):
    i = pl.program_id(0)

    @pl.when(i == 0)
    def _trunk():
        h1 = jnp.maximum(
            lax.dot_general(s_ref[...], w1_ref[...], _CONTRACT_LAST,
                            preferred_element_type=jnp.float32) + b1_ref[...],
            0.0)
        h2 = jnp.maximum(
            lax.dot_general(h1, w2_ref[...], _CONTRACT_LAST,
                            preferred_element_type=jnp.float32) + b2_ref[...],
            0.0)
        h_ref[...] = h2
        cache_ref[...] = lax.dot_general(
            h2, wc_ref[...], _CONTRACT_LAST,
            preferred_element_type=jnp.float32) + bc_ref[...]

    rec_ref[...] = lax.dot_general(
        h_ref[...], wr_ref[...], _CONTRACT_LAST,
        preferred_element_type=jnp.float32) + br_ref[...]


def kernel(s, W1, b1, W2, b2, Wc, bc, Wr, br):
    b1r = b1.reshape(1, _HID)
    b2r = b2.reshape(1, _HID)
    bcr = bc.reshape(1, _F)
    brr = br.reshape(1, _RTOT)

    cache, rec = pl.pallas_call(
        _body,
        grid=(_GRID,),
        in_specs=[
            pl.BlockSpec((_B, _STATE), lambda i: (0, 0)),
            pl.BlockSpec((_HID, _STATE), lambda i: (0, 0)),
            pl.BlockSpec((1, _HID), lambda i: (0, 0)),
            pl.BlockSpec((_HID, _HID), lambda i: (0, 0)),
            pl.BlockSpec((1, _HID), lambda i: (0, 0)),
            pl.BlockSpec((_F, _HID), lambda i: (0, 0)),
            pl.BlockSpec((1, _F), lambda i: (0, 0)),
            pl.BlockSpec((_BLK, _HID), lambda i: (i, 0)),
            pl.BlockSpec((1, _BLK), lambda i: (0, i)),
        ],
        out_specs=[
            pl.BlockSpec((_B, _F), lambda i: (0, 0)),
            pl.BlockSpec((_B, _BLK), lambda i: (0, i)),
        ],
        out_shape=[
            jax.ShapeDtypeStruct((_B, _F), jnp.float32),
            jax.ShapeDtypeStruct((_B, _RTOT), jnp.float32),
        ],
        scratch_shapes=[pltpu.VMEM((_B, _HID), jnp.float32)],
        compiler_params=pltpu.CompilerParams(
            dimension_semantics=("arbitrary",)),
    )(s, W1, b1r, W2, b2r, Wc, bcr, Wr, brr)

    return (cache, rec.reshape(_B, _V, _F))


# speedup vs baseline: 1.0871x; 1.0871x over previous
import jax
import jax.numpy as jnp
from jax import lax
from jax.experimental import pallas as pl
from jax.experimental.pallas import tpu as pltpu

_B = 32
_STATE = 256
_HID = 1024
_F = 1000
_V = 64
_RTOT = _V * _F
_BLK = 2560
_GRID = _RTOT // _BLK

_CONTRACT_LAST = (((1,), (1,)), ((), ()))


def _body(s_ref, w1_ref, b1_ref, w2_ref, b2_ref, wc_ref, bc_ref,
          wr_ref, br_ref, cache_ref, rec_ref, h_ref):
    i = pl.program_id(0)

    @pl.when(i == 0)
    def _trunk():
        h1 = jnp.maximum(
            lax.dot_general(s_ref[...], w1_ref[...], _CONTRACT_LAST,
                            preferred_element_type=jnp.float32) + b1_ref[...],
            0.0)
        h2 = jnp.maximum(
            lax.dot_general(h1, w2_ref[...], _CONTRACT_LAST,
                            preferred_element_type=jnp.float32) + b2_ref[...],
            0.0)
        h_ref[...] = h2
        cache_ref[...] = lax.dot_general(
            h2, wc_ref[...], _CONTRACT_LAST,
            preferred_element_type=jnp.float32) + bc_ref[...]

    rec_ref[...] = lax.dot_general(
        h_ref[...], wr_ref[...], _CONTRACT_LAST,
        preferred_element_type=jnp.float32) + br_ref[...]


def kernel(s, W1, b1, W2, b2, Wc, bc, Wr, br):
    b1r = b1.reshape(1, _HID)
    b2r = b2.reshape(1, _HID)
    bcr = bc.reshape(1, _F)
    brr = br.reshape(1, _RTOT)

    cache, rec = pl.pallas_call(
        _body,
        grid=(_GRID,),
        in_specs=[
            pl.BlockSpec((_B, _STATE), lambda i: (0, 0)),
            pl.BlockSpec((_HID, _STATE), lambda i: (0, 0)),
            pl.BlockSpec((1, _HID), lambda i: (0, 0)),
            pl.BlockSpec((_HID, _HID), lambda i: (0, 0)),
            pl.BlockSpec((1, _HID), lambda i: (0, 0)),
            pl.BlockSpec((_F, _HID), lambda i: (0, 0)),
            pl.BlockSpec((1, _F), lambda i: (0, 0)),
            pl.BlockSpec((_BLK, _HID), lambda i: (i, 0)),
            pl.BlockSpec((1, _BLK), lambda i: (0, i)),
        ],
        out_specs=[
            pl.BlockSpec((_B, _F), lambda i: (0, 0)),
            pl.BlockSpec((_B, _BLK), lambda i: (0, i)),
        ],
        out_shape=[
            jax.ShapeDtypeStruct((_B, _F), jnp.float32),
            jax.ShapeDtypeStruct((_B, _RTOT), jnp.float32),
        ],
        scratch_shapes=[pltpu.VMEM((_B, _HID), jnp.float32)],
        compiler_params=pltpu.CompilerParams(
            dimension_semantics=("arbitrary",)),
    )(s, W1, b1r, W2, b2r, Wc, bcr, Wr, brr)

    return (cache, rec.reshape(_B, _V, _F))


# speedup vs baseline: 1.0886x; 1.0014x over previous
import jax
import jax.numpy as jnp
from jax import lax
from jax.experimental import pallas as pl
from jax.experimental.pallas import tpu as pltpu

_B = 32
_STATE = 256
_HID = 1024
_F = 1000
_V = 64
_RTOT = _V * _F
_BLK = 3200
_HALF = _BLK // 2
_GRID = _RTOT // _BLK

_CONTRACT_LAST = (((1,), (1,)), ((), ()))


def _body(s_ref, w1_ref, b1_ref, w2_ref, b2_ref, wc_ref, bc_ref,
          wr_a_ref, wr_b_ref, br_ref, cache_ref, rec_ref, h_ref):
    i = pl.program_id(0)

    @pl.when(i == 0)
    def _trunk():
        h1 = jnp.maximum(
            lax.dot_general(s_ref[...], w1_ref[...], _CONTRACT_LAST,
                            preferred_element_type=jnp.float32) + b1_ref[...],
            0.0)
        h2 = jnp.maximum(
            lax.dot_general(h1, w2_ref[...], _CONTRACT_LAST,
                            preferred_element_type=jnp.float32) + b2_ref[...],
            0.0)
        h_ref[...] = h2
        cache_ref[...] = lax.dot_general(
            h2, wc_ref[...], _CONTRACT_LAST,
            preferred_element_type=jnp.float32) + bc_ref[...]

    h = h_ref[...]
    rec_ref[:, :_HALF] = lax.dot_general(
        h, wr_a_ref[...], _CONTRACT_LAST,
        preferred_element_type=jnp.float32) + br_ref[:, :_HALF]
    rec_ref[:, _HALF:] = lax.dot_general(
        h, wr_b_ref[...], _CONTRACT_LAST,
        preferred_element_type=jnp.float32) + br_ref[:, _HALF:]


def kernel(s, W1, b1, W2, b2, Wc, bc, Wr, br):
    b1r = b1.reshape(1, _HID)
    b2r = b2.reshape(1, _HID)
    bcr = bc.reshape(1, _F)
    brr = br.reshape(1, _RTOT)

    cache, rec = pl.pallas_call(
        _body,
        grid=(_GRID,),
        in_specs=[
            pl.BlockSpec((_B, _STATE), lambda i: (0, 0)),
            pl.BlockSpec((_HID, _STATE), lambda i: (0, 0)),
            pl.BlockSpec((1, _HID), lambda i: (0, 0)),
            pl.BlockSpec((_HID, _HID), lambda i: (0, 0)),
            pl.BlockSpec((1, _HID), lambda i: (0, 0)),
            pl.BlockSpec((_F, _HID), lambda i: (0, 0)),
            pl.BlockSpec((1, _F), lambda i: (0, 0)),
            pl.BlockSpec((_HALF, _HID), lambda i: (2 * i, 0)),
            pl.BlockSpec((_HALF, _HID), lambda i: (2 * i + 1, 0)),
            pl.BlockSpec((1, _BLK), lambda i: (0, i)),
        ],
        out_specs=[
            pl.BlockSpec((_B, _F), lambda i: (0, 0)),
            pl.BlockSpec((_B, _BLK), lambda i: (0, i)),
        ],
        out_shape=[
            jax.ShapeDtypeStruct((_B, _F), jnp.float32),
            jax.ShapeDtypeStruct((_B, _RTOT), jnp.float32),
        ],
        scratch_shapes=[pltpu.VMEM((_B, _HID), jnp.float32)],
        compiler_params=pltpu.CompilerParams(
            dimension_semantics=("arbitrary",)),
    )(s, W1, b1r, W2, b2r, Wc, bcr, Wr, Wr, brr)

    return (cache, rec.reshape(_B, _V, _F))
